# Initial kernel scaffold; baseline (speedup 1.0000x reference)
import jax
import jax.numpy as jnp
from jax.experimental import pallas as pl


def kernel(z, embedding):
    raise NotImplementedError("write your pallas kernel here")


# speedup vs baseline: 1.2567x; 1.2567x over previous
import jax
import jax.numpy as jnp
from jax.experimental import pallas as pl
from jax.experimental.pallas import tpu as pltpu

B = 8
C = 256
HW = 1024
K = 1024
N_TOTAL = B * C * HW


def _vq_kernel(z_ref, e_ref, q_ref, idx_ref, loss_ref, perp_ref,
               counts_ref, loss_acc):
    b = pl.program_id(0)
    z = z_ref[0]
    emb = e_ref[...]

    m = jnp.dot(emb, z, preferred_element_type=jnp.float32)
    z_sq = jnp.sum(z * z, axis=0, keepdims=True)
    e_sq = jnp.sum(emb * emb, axis=1, keepdims=True)
    dist = (z_sq + e_sq) - 2.0 * m

    mn = jnp.min(dist, axis=0, keepdims=True)
    iota_k = jax.lax.broadcasted_iota(jnp.int32, (K, HW), 0)
    idx2 = jnp.min(jnp.where(dist == mn, iota_k, jnp.int32(K)),
                   axis=0, keepdims=True)
    idx_ref[0] = idx2

    onehot = (iota_k == idx2).astype(jnp.float32)
    q = jax.lax.dot_general(emb, onehot, (((0,), (0,)), ((), ())),
                            preferred_element_type=jnp.float32)
    q_ref[0] = q

    diff = q - z
    sq = jnp.sum(diff * diff, keepdims=True).reshape(1, 1)
    cnt = jnp.sum(onehot, axis=1, keepdims=True)

    @pl.when(b == 0)
    def _init():
        loss_acc[...] = sq
        counts_ref[...] = cnt

    @pl.when(b != 0)
    def _accum():
        loss_acc[...] = loss_acc[...] + sq
        counts_ref[...] = counts_ref[...] + cnt

    @pl.when(b == B - 1)
    def _finalize():
        mse = loss_acc[...] * jnp.float32(1.0 / N_TOTAL)
        loss_ref[...] = mse + jnp.float32(0.25) * mse
        p = counts_ref[...] * jnp.float32(1.0 / (B * HW))
        ent = -jnp.sum(p * jnp.log(p + jnp.float32(1e-10)),
                       keepdims=True).reshape(1, 1)
        perp_ref[...] = jnp.exp(ent)


def kernel(z, embedding):
    z_r = z.reshape(B, C, HW)
    q, idx, loss, perp = pl.pallas_call(
        _vq_kernel,
        grid=(B,),
        in_specs=[
            pl.BlockSpec((1, C, HW), lambda b: (b, 0, 0)),
            pl.BlockSpec((K, C), lambda b: (0, 0)),
        ],
        out_specs=[
            pl.BlockSpec((1, C, HW), lambda b: (b, 0, 0)),
            pl.BlockSpec((1, 1, HW), lambda b: (b, 0, 0)),
            pl.BlockSpec((1, 1), lambda b: (0, 0)),
            pl.BlockSpec((1, 1), lambda b: (0, 0)),
        ],
        out_shape=[
            jax.ShapeDtypeStruct((B, C, HW), jnp.float32),
            jax.ShapeDtypeStruct((B, 1, HW), jnp.int32),
            jax.ShapeDtypeStruct((1, 1), jnp.float32),
            jax.ShapeDtypeStruct((1, 1), jnp.float32),
        ],
        scratch_shapes=[
            pltpu.VMEM((K, 1), jnp.float32),
            pltpu.VMEM((1, 1), jnp.float32),
        ],
        compiler_params=pltpu.CompilerParams(
            dimension_semantics=("arbitrary",)),
    )(z_r, embedding)
    return (q.reshape(z.shape), loss[0, 0], perp[0, 0], idx.reshape(-1))
